# baseline (speedup 1.0000x reference)
import functools
import math

import jax
import jax.numpy as jnp
from jax import lax
from jax.experimental import pallas as pl
from jax.experimental.pallas import tpu as pltpu
from jax.experimental.pallas import tpu_sc as plsc

D_MODEL = 64
SCALE = math.sqrt(D_MODEL)

NUM_CORES = 2
NUM_SUBCORES = 16
NUM_WORKERS = NUM_CORES * NUM_SUBCORES

SEQ = 50
BATCH = 16384
B_PER_W = BATCH // NUM_WORKERS
BLK = 128
SUBS = B_PER_W // BLK
TILE = 1024
NBUF = 2

_mesh = plsc.VectorSubcoreMesh(core_axis_name="c", subcore_axis_name="s")


@functools.partial(
    pl.kernel,
    mesh=_mesh,
    compiler_params=pltpu.CompilerParams(use_tc_tiling_on_sc=False, needs_layout_passes=False),
    out_type=jax.ShapeDtypeStruct((SEQ * (D_MODEL // 8) * (BATCH // BLK) * 8, BLK), jnp.float32),
    scratch_types=[
        pltpu.VMEM((B_PER_W * SEQ,), jnp.int32),
        pltpu.VMEM((B_PER_W // 2,), jnp.int32),
        pltpu.VMEM((B_PER_W // 2,), jnp.int32),
        pltpu.VMEM((B_PER_W // 2, 2 * D_MODEL), jnp.float32),
        pltpu.VMEM((B_PER_W // 2, 2 * D_MODEL), jnp.float32),
        pltpu.VMEM((D_MODEL, BLK + 1), jnp.float32),
        pltpu.VMEM((D_MODEL, BLK + 1), jnp.float32),
        pltpu.SemaphoreType.DMA((NBUF,)),
        pltpu.SemaphoreType.DMA((NBUF,)),
    ],
)
def _emb_lookup(x_hbm, table_hbm, out_hbm, xv, gidx0, gidx1, rows0, rows1,
                obuf0, obuf1, gsem, wsem):
    gidx = (gidx0, gidx1)
    rows = (rows0, rows1)
    obuf = (obuf0, obuf1)

    wid = lax.axis_index("s") * NUM_CORES + lax.axis_index("c")
    base = wid * (B_PER_W * SEQ)
    pltpu.sync_copy(x_hbm.at[pl.ds(base, B_PER_W * SEQ)], xv)

    iota = lax.iota(jnp.int32, 16)
    iota50 = iota * SEQ
    cidx = [iota + 16 * j0 for j0 in range(D_MODEL // 16)]

    def build_gidx(s, h, slot):
        g = gidx[slot]

        @plsc.parallel_loop(0, B_PER_W // 32, step=1, unroll=4)
        def _bg(j):
            idx16 = iota50 + ((h * (B_PER_W // 2) + 16 * j) * SEQ + s)
            g[pl.ds(16 * j, 16)] = plsc.load_gather(xv, [idx16])

    def issue_gather(slot):
        pltpu.async_copy(
            table_hbm.at[gidx[slot]], rows[slot], gsem.at[slot]
        )

    def wait_gather(slot):
        pltpu.make_async_copy(
            table_hbm.at[gidx[slot]], rows[slot], gsem.at[slot]
        ).wait()

    def issue_wb(s, h, bsub, ob):
        col = wid * SUBS + h * 2 + bsub
        for c8 in range(D_MODEL // 8):
            row0 = ((s * (D_MODEL // 8) + c8) * (BATCH // BLK) + col) * 8
            pltpu.async_copy(
                obuf[ob].at[pl.ds(c8 * 8, 8), pl.ds(0, BLK)],
                out_hbm.at[pl.ds(row0, 8), :],
                wsem.at[ob],
            )

    def wait_wb(ob):
        pltpu.make_async_copy(
            obuf[ob].at[pl.ds(0, D_MODEL), pl.ds(0, BLK)],
            out_hbm.at[pl.ds(0, D_MODEL), :],
            wsem.at[ob],
        ).wait()

    def transpose_scale(slot, bsub, ob):
        r = rows[slot]
        o = obuf[ob]

        @plsc.parallel_loop(0, BLK, step=1, unroll=8)
        def _tr(bl):
            blvec = jnp.full((16,), bl, dtype=jnp.int32)
            for j0 in range(D_MODEL // 16):
                v = r[bsub * BLK + bl, pl.ds(16 * j0, 16)] * SCALE
                plsc.store_scatter(o, [cidx[j0], blvec], v)

    build_gidx(0, 0, 0)
    issue_gather(0)

    def outer(g, carry):
        for k in range(NBUF):
            slot = k
            nslot = (k + 1) % NBUF
            wait_gather(slot)

            if k == 0:
                build_gidx(g, 1, nslot)
                issue_gather(nslot)
            else:

                @pl.when(g < SEQ - 1)
                def _():
                    build_gidx(g + 1, 0, nslot)
                    issue_gather(nslot)

            for bsub in range(2):
                ob = bsub
                if k == 0:

                    @pl.when(g > 0)
                    def _():
                        wait_wb(ob)

                else:
                    wait_wb(ob)
                transpose_scale(slot, bsub, ob)
                issue_wb(g, k, bsub, ob)
        return carry

    lax.fori_loop(0, SEQ, outer, 0)
    for ob in range(2):
        wait_wb(ob)


def kernel(x, table):
    flat = x.reshape(-1)
    t2 = jnp.concatenate([table, table], axis=1)
    out5 = _emb_lookup(flat, t2)
    out = out5.reshape(SEQ, D_MODEL // 8, BATCH // BLK, 8, BLK)
    out = out.transpose(2, 4, 0, 1, 3).reshape(BATCH, SEQ, D_MODEL)
    return out


# speedup vs baseline: 1.2137x; 1.2137x over previous
import functools
import math

import jax
import jax.numpy as jnp
from jax import lax
from jax.experimental import pallas as pl
from jax.experimental.pallas import tpu as pltpu
from jax.experimental.pallas import tpu_sc as plsc

D_MODEL = 64
SCALE = math.sqrt(D_MODEL)

NUM_CORES = 2
NUM_SUBCORES = 16
NUM_WORKERS = NUM_CORES * NUM_SUBCORES

SEQ = 50
BATCH = 16384
B_PER_W = BATCH // NUM_WORKERS
BLK = 128
SUBS = B_PER_W // BLK
TILE = 1024
NBUF = 2

_mesh = plsc.VectorSubcoreMesh(core_axis_name="c", subcore_axis_name="s")


@functools.partial(
    pl.kernel,
    mesh=_mesh,
    compiler_params=pltpu.CompilerParams(use_tc_tiling_on_sc=False, needs_layout_passes=False),
    out_type=jax.ShapeDtypeStruct((SEQ * (D_MODEL // 8) * (BATCH // BLK) * 8, BLK), jnp.float32),
    scratch_types=[
        pltpu.VMEM((B_PER_W * SEQ,), jnp.int32),
        pltpu.VMEM((B_PER_W,), jnp.int32),
        pltpu.VMEM((B_PER_W,), jnp.int32),
        pltpu.VMEM((B_PER_W, D_MODEL), jnp.float32),
        pltpu.VMEM((B_PER_W, D_MODEL), jnp.float32),
        pltpu.VMEM((D_MODEL, BLK + 1), jnp.float32),
        pltpu.VMEM((D_MODEL, BLK + 1), jnp.float32),
        pltpu.SemaphoreType.DMA((NBUF,)),
        pltpu.SemaphoreType.DMA((NBUF,)),
    ],
)
def _emb_lookup(x_hbm, table_hbm, out_hbm, xv, gidx0, gidx1, rows0, rows1,
                obuf0, obuf1, gsem, wsem):
    gidx = (gidx0, gidx1)
    rows = (rows0, rows1)
    obuf = (obuf0, obuf1)

    wid = lax.axis_index("s") * NUM_CORES + lax.axis_index("c")
    base = wid * (B_PER_W * SEQ)
    pltpu.sync_copy(x_hbm.at[pl.ds(base, B_PER_W * SEQ)], xv)

    iota = lax.iota(jnp.int32, 16)
    iota50 = iota * SEQ
    cidx = [iota + 16 * j0 for j0 in range(D_MODEL // 16)]

    def build_gidx(s, slot):
        g = gidx[slot]

        @plsc.parallel_loop(0, B_PER_W // 16, step=1, unroll=8)
        def _bg(j):
            idx16 = iota50 + (16 * j * SEQ + s)
            g[pl.ds(16 * j, 16)] = plsc.load_gather(xv, [idx16])

    def issue_gather(slot):
        pltpu.async_copy(
            table_hbm.at[gidx[slot]], rows[slot], gsem.at[slot]
        )

    def wait_gather(slot):
        pltpu.make_async_copy(
            table_hbm.at[gidx[slot]], rows[slot], gsem.at[slot]
        ).wait()

    def issue_wb(s, bsub, ob):
        col = wid * SUBS + bsub
        for c8 in range(D_MODEL // 8):
            row0 = ((s * (D_MODEL // 8) + c8) * (BATCH // BLK) + col) * 8
            pltpu.async_copy(
                obuf[ob].at[pl.ds(c8 * 8, 8), pl.ds(0, BLK)],
                out_hbm.at[pl.ds(row0, 8), :],
                wsem.at[ob],
            )

    def wait_wb(ob):
        pltpu.make_async_copy(
            obuf[ob].at[pl.ds(0, D_MODEL), pl.ds(0, BLK)],
            out_hbm.at[pl.ds(0, D_MODEL), :],
            wsem.at[ob],
        ).wait()

    def transpose_scale(slot, bsub, ob):
        r = rows[slot]
        o = obuf[ob]

        @plsc.parallel_loop(0, BLK, step=1, unroll=16)
        def _tr(bl):
            blvec = jnp.full((16,), bl, dtype=jnp.int32)
            for j0 in range(D_MODEL // 16):
                v = r[bsub * BLK + bl, pl.ds(16 * j0, 16)] * SCALE
                plsc.store_scatter(o, [cidx[j0], blvec], v)

    build_gidx(0, 0)
    issue_gather(0)

    def outer(g, carry):
        for k in range(NBUF):
            s = g * NBUF + k
            slot = k
            nslot = (k + 1) % NBUF
            wait_gather(slot)

            if k == 0:
                build_gidx(s + 1, nslot)
                issue_gather(nslot)
            else:

                @pl.when(g < SEQ // NBUF - 1)
                def _():
                    build_gidx(s + 1, nslot)
                    issue_gather(nslot)

            for bsub in range(SUBS):
                ob = bsub % 2
                if k == 0 and bsub < 2:

                    @pl.when(g > 0)
                    def _():
                        wait_wb(ob)

                else:
                    wait_wb(ob)
                transpose_scale(slot, bsub, ob)
                issue_wb(s, bsub, ob)
        return carry

    lax.fori_loop(0, SEQ // NBUF, outer, 0)
    for ob in range(2):
        wait_wb(ob)


def kernel(x, table):
    flat = x.reshape(-1)
    out5 = _emb_lookup(flat, table)
    out = out5.reshape(SEQ, D_MODEL // 8, BATCH // BLK, 8, BLK)
    out = out.transpose(2, 4, 0, 1, 3).reshape(BATCH, SEQ, D_MODEL)
    return out


# speedup vs baseline: 1.2501x; 1.0300x over previous
import functools
import math

import jax
import jax.numpy as jnp
from jax import lax
from jax.experimental import pallas as pl
from jax.experimental.pallas import tpu as pltpu
from jax.experimental.pallas import tpu_sc as plsc

D_MODEL = 64
SCALE = math.sqrt(D_MODEL)

NUM_CORES = 2
NUM_SUBCORES = 16
NUM_WORKERS = NUM_CORES * NUM_SUBCORES

SEQ = 50
BATCH = 16384
B_PER_W = BATCH // NUM_WORKERS
BLK = 128
SUBS = B_PER_W // BLK
TILE = 1024
NBUF = 2

_mesh = plsc.VectorSubcoreMesh(core_axis_name="c", subcore_axis_name="s")


@functools.partial(
    pl.kernel,
    mesh=_mesh,
    compiler_params=pltpu.CompilerParams(use_tc_tiling_on_sc=False, needs_layout_passes=False),
    out_type=jax.ShapeDtypeStruct((SEQ * (D_MODEL // 8) * (BATCH // BLK) * 8, BLK), jnp.float32),
    scratch_types=[
        pltpu.VMEM((B_PER_W * SEQ,), jnp.int32),
        pltpu.VMEM((B_PER_W,), jnp.int32),
        pltpu.VMEM((B_PER_W,), jnp.int32),
        pltpu.VMEM((B_PER_W, D_MODEL), jnp.float32),
        pltpu.VMEM((B_PER_W, D_MODEL), jnp.float32),
        pltpu.VMEM((D_MODEL, BLK + 1), jnp.float32),
        pltpu.VMEM((D_MODEL, BLK + 1), jnp.float32),
        pltpu.SemaphoreType.DMA((NBUF,)),
        pltpu.SemaphoreType.DMA((NBUF,)),
    ],
)
def _emb_lookup(x_hbm, table_hbm, out_hbm, xv, gidx0, gidx1, rows0, rows1,
                obuf0, obuf1, gsem, wsem):
    gidx = (gidx0, gidx1)
    rows = (rows0, rows1)
    obuf = (obuf0, obuf1)

    wid = lax.axis_index("s") * NUM_CORES + lax.axis_index("c")
    base = wid * (B_PER_W * SEQ)
    pltpu.sync_copy(x_hbm.at[pl.ds(base, B_PER_W * SEQ)], xv)

    iota = lax.iota(jnp.int32, 16)
    iota50 = iota * SEQ
    cidx = [iota + 16 * j0 for j0 in range(D_MODEL // 16)]

    def build_gidx(s, slot):
        g = gidx[slot]

        @plsc.parallel_loop(0, B_PER_W // 16, step=1, unroll=4)
        def _bg(j):
            idx16 = iota50 + (16 * j * SEQ + s)
            g[pl.ds(16 * j, 16)] = plsc.load_gather(xv, [idx16])

    def issue_gather(slot):
        pltpu.async_copy(
            table_hbm.at[gidx[slot]], rows[slot], gsem.at[slot]
        )

    def wait_gather(slot):
        pltpu.make_async_copy(
            table_hbm.at[gidx[slot]], rows[slot], gsem.at[slot]
        ).wait()

    def issue_wb(s, bsub, ob):
        col = wid * SUBS + bsub
        for c8 in range(D_MODEL // 8):
            row0 = ((s * (D_MODEL // 8) + c8) * (BATCH // BLK) + col) * 8
            pltpu.async_copy(
                obuf[ob].at[pl.ds(c8 * 8, 8), pl.ds(0, BLK)],
                out_hbm.at[pl.ds(row0, 8), :],
                wsem.at[ob],
            )

    def wait_wb(ob):
        pltpu.make_async_copy(
            obuf[ob].at[pl.ds(0, D_MODEL), pl.ds(0, BLK)],
            out_hbm.at[pl.ds(0, D_MODEL), :],
            wsem.at[ob],
        ).wait()

    def transpose_scale(slot, bsub, ob):
        r = rows[slot]
        o = obuf[ob]

        @plsc.parallel_loop(0, BLK, step=1, unroll=8)
        def _tr(bl):
            blvec = jnp.full((16,), bl, dtype=jnp.int32)
            for j0 in range(D_MODEL // 16):
                v = r[bsub * BLK + bl, pl.ds(16 * j0, 16)] * SCALE
                plsc.store_scatter(o, [cidx[j0], blvec], v)

    build_gidx(0, 0)
    issue_gather(0)

    def outer(g, carry):
        for k in range(NBUF):
            s = g * NBUF + k
            slot = k
            nslot = (k + 1) % NBUF
            wait_gather(slot)

            if k == 0:
                build_gidx(s + 1, nslot)
                issue_gather(nslot)
            else:

                @pl.when(g < SEQ // NBUF - 1)
                def _():
                    build_gidx(s + 1, nslot)
                    issue_gather(nslot)

            for bsub in range(SUBS):
                ob = bsub % 2
                if k == 0 and bsub < 2:

                    @pl.when(g > 0)
                    def _():
                        wait_wb(ob)

                else:
                    wait_wb(ob)
                transpose_scale(slot, bsub, ob)
                issue_wb(s, bsub, ob)
        return carry

    lax.fori_loop(0, SEQ // NBUF, outer, 0)
    for ob in range(2):
        wait_wb(ob)


def kernel(x, table):
    flat = x.reshape(-1)
    out5 = _emb_lookup(flat, table)
    out = out5.reshape(SEQ, D_MODEL // 8, BATCH // BLK, 8, BLK)
    out = out.transpose(2, 4, 0, 1, 3).reshape(BATCH, SEQ, D_MODEL)
    return out
